# Initial kernel scaffold; baseline (speedup 1.0000x reference)
import jax
import jax.numpy as jnp
from jax.experimental import pallas as pl


def kernel(x, edge_index, edge_attr, W1, b1, W2, b2):
    raise NotImplementedError("write your pallas kernel here")


# speedup vs baseline: 23.9430x; 23.9430x over previous
import functools

import jax
import jax.numpy as jnp
from jax import lax
from jax.experimental import pallas as pl
from jax.experimental.pallas import tpu as pltpu
from jax.experimental.pallas import tpu_sc as plsc

N = 10000
NP = 10240
D = 128
E = 320000
NC = 2
NS = 16
NW = NC * NS
EPT = E // NW
CH = 80
NCH = EPT // CH
RPT = NP // NS
NB = 2
SI = 25
NST = NCH // SI

_mesh = plsc.VectorSubcoreMesh(
    core_axis_name="c", subcore_axis_name="s", num_cores=NC, num_subcores=NS)


@functools.partial(
    pl.kernel,
    out_type=jax.ShapeDtypeStruct((NC, NP), jnp.float32),
    mesh=_mesh,
    scratch_types=[
        pltpu.VMEM((NST, SI, CH), jnp.int32),
        pltpu.VMEM((CH,), jnp.float32),
        pltpu.VMEM((RPT,), jnp.float32),
        pltpu.VMEM_SHARED((NP,), jnp.float32),
    ],
)
def _sc_count(dst_h, cnt_h, idx_v, ones_v, zv, cnt_sh):
    cid = lax.axis_index("c")
    sid = lax.axis_index("s")
    wid = cid * NS + sid
    for i in range(CH // 16):
        ones_v[pl.ds(i * 16, 16)] = jnp.ones((16,), jnp.float32)
    for i in range(RPT // 16):
        zv[pl.ds(i * 16, 16)] = jnp.zeros((16,), jnp.float32)
    pltpu.sync_copy(zv, cnt_sh.at[pl.ds(sid * RPT, RPT)])
    plsc.subcore_barrier()
    pltpu.sync_copy(dst_h.at[wid], idx_v)

    def chunk(c, carry):
        pltpu.sync_copy(ones_v, cnt_sh.at[idx_v.at[c // SI, c % SI]], add=True)
        return carry

    lax.fori_loop(0, NCH, chunk, 0)
    plsc.subcore_barrier()
    pltpu.sync_copy(cnt_sh.at[pl.ds(sid * RPT, RPT)],
                    cnt_h.at[cid, pl.ds(sid * RPT, RPT)])


@functools.partial(
    pl.kernel,
    out_type=jax.ShapeDtypeStruct((NC, NP, D), jnp.float32),
    mesh=_mesh,
    scratch_types=[
        pltpu.VMEM((SI, CH), jnp.int32),
        pltpu.VMEM((SI, CH), jnp.int32),
        pltpu.VMEM((NB, CH, D), jnp.float32),
        pltpu.VMEM_SHARED((NP, D), jnp.float32),
        pltpu.SemaphoreType.DMA,
    ],
)
def _sc_agg(src_h, dst_h, g_h, acc_h, isrc, idst, rows, acc_sh, sem):
    cid = lax.axis_index("c")
    sid = lax.axis_index("s")
    wid = cid * NS + sid

    def zbody(i, carry):
        for j in range(D // 16):
            rows[0, i, pl.ds(j * 16, 16)] = jnp.zeros((16,), jnp.float32)
        return carry

    lax.fori_loop(0, CH, zbody, 0)
    for k in range(RPT // CH):
        pltpu.sync_copy(rows.at[0], acc_sh.at[pl.ds(sid * RPT + k * CH, CH)])
    plsc.subcore_barrier()

    def stage(st, carry):
        pltpu.sync_copy(src_h.at[wid, st], isrc)
        pltpu.sync_copy(dst_h.at[wid, st], idst)

        def chunk(c, carry2):
            pltpu.async_copy(g_h.at[isrc.at[c]], rows.at[0], sem).wait()
            pltpu.sync_copy(rows.at[0], acc_sh.at[idst.at[c]], add=True)
            return carry2

        lax.fori_loop(0, SI, chunk, 0)
        return carry

    lax.fori_loop(0, NST, stage, 0)
    plsc.subcore_barrier()
    pltpu.sync_copy(acc_sh.at[pl.ds(sid * RPT, RPT)],
                    acc_h.at[cid, pl.ds(sid * RPT, RPT)])


@functools.partial(
    pl.kernel,
    out_type=jax.ShapeDtypeStruct((NC, NP), jnp.float32),
    mesh=_mesh,
    scratch_types=[
        pltpu.VMEM((NST, SI, CH), jnp.int32),
        pltpu.VMEM((NST, SI, CH), jnp.int32),
        pltpu.VMEM((CH,), jnp.float32),
        pltpu.VMEM((RPT,), jnp.float32),
        pltpu.VMEM_SHARED((NP,), jnp.float32),
        pltpu.SemaphoreType.DMA,
    ],
)
def _sc_agg1(src_h, dst_h, t_h, acc_h, isrc, idst, tv, zv, acc_sh, sem):
    cid = lax.axis_index("c")
    sid = lax.axis_index("s")
    wid = cid * NS + sid
    for i in range(RPT // 16):
        zv[pl.ds(i * 16, 16)] = jnp.zeros((16,), jnp.float32)
    pltpu.sync_copy(zv, acc_sh.at[pl.ds(sid * RPT, RPT)])
    plsc.subcore_barrier()
    pltpu.sync_copy(src_h.at[wid], isrc)
    pltpu.sync_copy(dst_h.at[wid], idst)

    def chunk(c, carry):
        pltpu.async_copy(t_h.at[isrc.at[c // SI, c % SI]], tv, sem).wait()
        pltpu.sync_copy(tv, acc_sh.at[idst.at[c // SI, c % SI]], add=True)
        return carry

    lax.fori_loop(0, NCH, chunk, 0)
    plsc.subcore_barrier()
    pltpu.sync_copy(acc_sh.at[pl.ds(sid * RPT, RPT)],
                    acc_h.at[cid, pl.ds(sid * RPT, RPT)])


BM = 640
GRID = NP // BM


def _tc_b_body(x_ref, w1_ref, c0_ref, c1_ref, g_ref, dinv_ref):
    cnt = c0_ref[...] + c1_ref[...]
    dinv = lax.rsqrt(cnt + 1.0)
    h = jnp.dot(x_ref[...], w1_ref[...], preferred_element_type=jnp.float32)
    g_ref[...] = h * dinv
    dinv_ref[...] = dinv


def _tc_b(xp, w1, c0, c1):
    return pl.pallas_call(
        _tc_b_body,
        grid=(GRID,),
        in_specs=[
            pl.BlockSpec((BM, D), lambda i: (i, 0)),
            pl.BlockSpec((D, D), lambda i: (0, 0)),
            pl.BlockSpec((BM, 1), lambda i: (i, 0)),
            pl.BlockSpec((BM, 1), lambda i: (i, 0)),
        ],
        out_specs=[
            pl.BlockSpec((BM, D), lambda i: (i, 0)),
            pl.BlockSpec((BM, 1), lambda i: (i, 0)),
        ],
        out_shape=[
            jax.ShapeDtypeStruct((NP, D), jnp.float32),
            jax.ShapeDtypeStruct((NP, 1), jnp.float32),
        ],
    )(xp, w1, c0, c1)


def _tc_d_body(a0_ref, a1_ref, g_ref, dinv_ref, b1_ref, w2_ref, t_ref):
    acc = a0_ref[...] + a1_ref[...] + g_ref[...]
    out1 = acc * dinv_ref[...] + b1_ref[...]
    h1 = jnp.maximum(out1, 0.0)
    s = jnp.dot(h1, w2_ref[...], preferred_element_type=jnp.float32)
    t_ref[...] = s * dinv_ref[...]


def _tc_d(a0, a1, g, dinv, b1r, w2):
    return pl.pallas_call(
        _tc_d_body,
        grid=(GRID,),
        in_specs=[
            pl.BlockSpec((BM, D), lambda i: (i, 0)),
            pl.BlockSpec((BM, D), lambda i: (i, 0)),
            pl.BlockSpec((BM, D), lambda i: (i, 0)),
            pl.BlockSpec((BM, 1), lambda i: (i, 0)),
            pl.BlockSpec((1, D), lambda i: (0, 0)),
            pl.BlockSpec((D, 1), lambda i: (0, 0)),
        ],
        out_specs=pl.BlockSpec((BM, 1), lambda i: (i, 0)),
        out_shape=jax.ShapeDtypeStruct((NP, 1), jnp.float32),
    )(a0, a1, g, dinv, b1r, w2)


def _tc_f_body(q0_ref, q1_ref, t_ref, dinv_ref, b2_ref, o_ref):
    o_ref[...] = (q0_ref[...] + q1_ref[...] + t_ref[...]) * dinv_ref[...] \
        + b2_ref[...]


def _tc_f(q0, q1, t, dinv, b2r):
    return pl.pallas_call(
        _tc_f_body,
        grid=(GRID,),
        in_specs=[
            pl.BlockSpec((BM, 1), lambda i: (i, 0)),
            pl.BlockSpec((BM, 1), lambda i: (i, 0)),
            pl.BlockSpec((BM, 1), lambda i: (i, 0)),
            pl.BlockSpec((BM, 1), lambda i: (i, 0)),
            pl.BlockSpec((1, 1), lambda i: (0, 0)),
        ],
        out_specs=pl.BlockSpec((BM, 1), lambda i: (i, 0)),
        out_shape=jax.ShapeDtypeStruct((NP, 1), jnp.float32),
    )(q0, q1, t, dinv, b2r)


def kernel(x, edge_index, edge_attr, W1, b1, W2, b2):
    ei = edge_index.astype(jnp.int32)
    src = ei[0].reshape(NW, NST, SI, CH)
    dst = ei[1].reshape(NW, NST, SI, CH)
    xp = jnp.zeros((NP, D), jnp.float32).at[:N].set(x)

    cnt = _sc_count(dst)
    c0 = cnt[0].reshape(NP, 1)
    c1 = cnt[1].reshape(NP, 1)
    g, dinv = _tc_b(xp, W1, c0, c1)
    acc = _sc_agg(src, dst, g)
    t = _tc_d(acc[0], acc[1], g, dinv, b1.reshape(1, D), W2)
    q = _sc_agg1(src, dst, t.reshape(NP))
    out = _tc_f(q[0].reshape(NP, 1), q[1].reshape(NP, 1), t, dinv,
                b2.reshape(1, 1))
    return out[:N]
